# baseline (speedup 1.0000x reference)
import functools

import jax
import jax.numpy as jnp
from jax.experimental import pallas as pl
from jax.experimental.pallas import tpu as pltpu

_TQ = 256
_KB = 4096
_BIGF = float(2 ** 24)
_TOP_K = 5
_NUM_CLASSES = 3
_NCAND = 256


def _phase1_body(x_ref, keys_ref, qsq_ref, ksq_ref, lp_ref,
                 d5_ref, p5_ref, *, tq, kb):
    k = pl.program_id(1)
    ns = kb // 128

    x = x_ref[...]
    ks = keys_ref[...]
    m = jax.lax.dot_general(x, ks, (((1,), (1,)), ((), ())),
                            preferred_element_type=jnp.float32)
    d = (qsq_ref[...] + ksq_ref[0]) - 2.0 * m

    slices = [d[:, s * 128:(s + 1) * 128] for s in range(ns)]
    items = [(sv, jnp.full((tq, 128), float(s), jnp.float32))
             for s, sv in enumerate(slices)]
    while len(items) > 1:
        nxt = []
        for i in range(0, len(items) - 1, 2):
            (av, aw), (bv_, bw_) = items[i], items[i + 1]
            le = av <= bv_
            nxt.append((jnp.minimum(av, bv_), jnp.where(le, aw, bw_)))
        if len(items) % 2:
            nxt.append(items[-1])
        items = nxt
    rv, rw = items[0]

    lanef = jax.lax.broadcasted_iota(jnp.int32, (tq, 128), 1).astype(
        jnp.float32)
    rp2 = rw * 128.0 + lanef

    pmrs = []
    for r in range(_TOP_K):
        mr = jnp.min(rv, axis=1, keepdims=True)
        pmr = jnp.min(jnp.where(rv == mr, rp2, float(kb)), axis=1,
                      keepdims=True)
        pmrs.append(pmr)
        if r < _TOP_K - 1:
            rv = jnp.where(rp2 == pmr, jnp.inf, rv)
    pm5 = jnp.concatenate(pmrs, axis=1)
    j5 = pm5.astype(jnp.int32) & 127
    j5f = j5.astype(jnp.float32)

    cand_d = [jnp.take_along_axis(sv, j5, axis=1, mode="promise_in_bounds")
              for sv in slices]
    lp0 = jnp.broadcast_to(lp_ref[0, :, :128], (tq, 128))
    lp1 = jnp.broadcast_to(lp_ref[0, :, 128:], (tq, 128))
    lab_a = jnp.take_along_axis(lp0, j5, axis=1, mode="promise_in_bounds")
    lab_b = jnp.take_along_axis(lp1, j5, axis=1, mode="promise_in_bounds")

    nc = _TOP_K * ns
    basef = (k * (4 * kb)).astype(jnp.float32)
    iota = jax.lax.broadcasted_iota(jnp.int32, (tq, nc), 1)
    sfl = (iota * 26215) >> 17
    wfl = iota - 5 * sfl
    tile_j = jnp.take_along_axis(j5f, wfl, axis=1,
                                 mode="promise_in_bounds")
    tile_la = jnp.take_along_axis(lab_a, wfl, axis=1,
                                  mode="promise_in_bounds")
    tile_lb = jnp.take_along_axis(lab_b, wfl, axis=1,
                                  mode="promise_in_bounds")
    sh_a = jnp.minimum(2 * sfl, 31)
    sh_b = jnp.maximum(2 * (sfl - 16), 0)
    lab_c = jnp.where(
        sfl < 16,
        jax.lax.shift_right_logical(tile_la, sh_a) & 3,
        jax.lax.shift_right_logical(tile_lb, sh_b) & 3).astype(jnp.float32)
    cand_p = (basef + (512.0 * sfl.astype(jnp.float32))
              + 4.0 * tile_j + lab_c)

    padd = jnp.full((tq, _NCAND - nc), jnp.inf, jnp.float32)
    padp = jnp.full((tq, _NCAND - nc), _BIGF, jnp.float32)
    d5_ref[...] = jnp.concatenate(cand_d + [padd], axis=1)[:, None, None, :]
    p5_ref[...] = jnp.concatenate([cand_p, padp], axis=1)[:, None, None, :]


def _phase2_body(d5_ref, p5_ref, out_ref, *, tq, nkb):
    nl = nkb * _NCAND
    dd = d5_ref[...].reshape(tq, nl)
    pp = p5_ref[...].reshape(tq, nl)

    ns2 = nl // 128
    items = [(dd[:, t * 128:(t + 1) * 128], pp[:, t * 128:(t + 1) * 128])
             for t in range(ns2)]
    d_slices = [it[0] for it in items]
    p_slices = [it[1] for it in items]
    while len(items) > 1:
        nxt = []
        for i in range(0, len(items) - 1, 2):
            (av, ap), (bv_, bp_) = items[i], items[i + 1]
            le = av <= bv_
            nxt.append((jnp.minimum(av, bv_), jnp.where(le, ap, bp_)))
        if len(items) % 2:
            nxt.append(items[-1])
        items = nxt
    rv, rp = items[0]

    lane = jax.lax.broadcasted_iota(jnp.int32, (tq, 128), 1).astype(
        jnp.float32)
    jcols = []
    for r in range(_TOP_K):
        mr = jnp.min(rv, axis=1, keepdims=True)
        pmr = jnp.min(jnp.where(rv == mr, rp, _BIGF), axis=1, keepdims=True)
        hit = rp == pmr
        jcols.append(jnp.min(jnp.where(hit, lane, 128.0), axis=1,
                             keepdims=True))
        if r < _TOP_K - 1:
            rv = jnp.where(hit, jnp.inf, rv)
    j5 = jnp.concatenate(jcols, axis=1).astype(jnp.int32)

    allv = jnp.concatenate(
        [jnp.take_along_axis(sv, j5, axis=1, mode="promise_in_bounds")
         for sv in d_slices], axis=1)
    allp = jnp.concatenate(
        [jnp.take_along_axis(sp_, j5, axis=1, mode="promise_in_bounds")
         for sp_ in p_slices], axis=1)
    np_ = []
    for r in range(_TOP_K):
        mr = jnp.min(allv, axis=1, keepdims=True)
        pmr = jnp.min(jnp.where(allv == mr, allp, _BIGF), axis=1,
                      keepdims=True)
        np_.append(pmr)
        if r < _TOP_K - 1:
            allv = jnp.where(allp == pmr, jnp.inf, allv)

    lab = jnp.concatenate(np_, axis=1).astype(jnp.int32) & 3
    cols = [jnp.sum((lab == c).astype(jnp.float32), axis=1, keepdims=True)
            for c in range(_NUM_CLASSES)]
    out_ref[...] = jnp.concatenate(cols, axis=1)


def kernel(x4, keys, labels):
    b, dmodel = x4.shape
    kn = keys.shape[0]
    tq = _TQ if b % _TQ == 0 else b
    kb = _KB
    kp = ((kn + kb - 1) // kb) * kb
    nkb = kp // kb
    nqt = b // tq
    ns = kb // 128

    q_sq = jnp.sum(x4 * x4, axis=1, keepdims=True)
    k_sq = jnp.sum(keys * keys, axis=1)
    pad = kp - kn
    k_sq_p = jnp.concatenate(
        [k_sq, jnp.full((pad,), jnp.inf, jnp.float32)]).reshape(nkb, 1, kb)
    keys_p = jnp.concatenate(
        [keys, jnp.zeros((pad, dmodel), jnp.float32)], axis=0)
    labels_p = jnp.concatenate(
        [labels, jnp.zeros((pad,), jnp.int32)]).reshape(nkb, ns, 128)
    shifts = (2 * (jnp.arange(16, dtype=jnp.int32)))[None, :, None]
    lp0 = jnp.sum(labels_p[:, :16] << shifts, axis=1, dtype=jnp.int32)
    lp1 = jnp.sum(labels_p[:, 16:] << shifts, axis=1, dtype=jnp.int32)
    lp = jnp.concatenate([lp0, lp1], axis=1).reshape(nkb, 1, 256)

    p1 = functools.partial(_phase1_body, tq=tq, kb=kb)
    d5, p5 = pl.pallas_call(
        p1,
        grid=(nqt, nkb),
        in_specs=[
            pl.BlockSpec((tq, dmodel), lambda q, k: (q, 0)),
            pl.BlockSpec((kb, dmodel), lambda q, k: (k, 0)),
            pl.BlockSpec((tq, 1), lambda q, k: (q, 0)),
            pl.BlockSpec((1, 1, kb), lambda q, k: (k, 0, 0)),
            pl.BlockSpec((1, 1, 256), lambda q, k: (k, 0, 0)),
        ],
        out_specs=[
            pl.BlockSpec((tq, 1, 1, _NCAND), lambda q, k: (q, k, 0, 0)),
            pl.BlockSpec((tq, 1, 1, _NCAND), lambda q, k: (q, k, 0, 0)),
        ],
        out_shape=[
            jax.ShapeDtypeStruct((b, nkb, 1, _NCAND), jnp.float32),
            jax.ShapeDtypeStruct((b, nkb, 1, _NCAND), jnp.float32),
        ],
        compiler_params=pltpu.CompilerParams(
            dimension_semantics=("parallel", "parallel")),
    )(x4, keys_p, q_sq, k_sq_p, lp)

    tq2 = 128 if b % 128 == 0 else b
    p2 = functools.partial(_phase2_body, tq=tq2, nkb=nkb)
    out = pl.pallas_call(
        p2,
        grid=(b // tq2,),
        in_specs=[
            pl.BlockSpec((tq2, nkb, 1, _NCAND), lambda q: (q, 0, 0, 0)),
            pl.BlockSpec((tq2, nkb, 1, _NCAND), lambda q: (q, 0, 0, 0)),
        ],
        out_specs=pl.BlockSpec((tq2, _NUM_CLASSES), lambda q: (q, 0)),
        out_shape=jax.ShapeDtypeStruct((b, _NUM_CLASSES), jnp.float32),
        compiler_params=pltpu.CompilerParams(
            dimension_semantics=("arbitrary",)),
    )(d5, p5)
    return out


# speedup vs baseline: 1.0007x; 1.0007x over previous
import functools

import jax
import jax.numpy as jnp
from jax.experimental import pallas as pl
from jax.experimental.pallas import tpu as pltpu

_TQ = 256
_KB = 4096
_BIGF = float(2 ** 24)
_TOP_K = 5
_NUM_CLASSES = 3
_NCAND = 256


def _phase1_body(x_ref, keys_ref, qsq_ref, ksq_ref, lp_ref,
                 d5_ref, p5_ref, *, tq, kb):
    k = pl.program_id(0)
    ns = kb // 128

    x = x_ref[...]
    ks = keys_ref[...]
    m = jax.lax.dot_general(x, ks, (((1,), (1,)), ((), ())),
                            preferred_element_type=jnp.float32)
    d = (qsq_ref[...] + ksq_ref[0]) - 2.0 * m

    slices = [d[:, s * 128:(s + 1) * 128] for s in range(ns)]
    items = [(sv, jnp.full((tq, 128), float(s), jnp.float32))
             for s, sv in enumerate(slices)]
    while len(items) > 1:
        nxt = []
        for i in range(0, len(items) - 1, 2):
            (av, aw), (bv_, bw_) = items[i], items[i + 1]
            le = av <= bv_
            nxt.append((jnp.minimum(av, bv_), jnp.where(le, aw, bw_)))
        if len(items) % 2:
            nxt.append(items[-1])
        items = nxt
    rv, rw = items[0]

    lanef = jax.lax.broadcasted_iota(jnp.int32, (tq, 128), 1).astype(
        jnp.float32)
    rp2 = rw * 128.0 + lanef

    pmrs = []
    for r in range(_TOP_K):
        mr = jnp.min(rv, axis=1, keepdims=True)
        pmr = jnp.min(jnp.where(rv == mr, rp2, float(kb)), axis=1,
                      keepdims=True)
        pmrs.append(pmr)
        if r < _TOP_K - 1:
            rv = jnp.where(rp2 == pmr, jnp.inf, rv)
    pm5 = jnp.concatenate(pmrs, axis=1)
    j5 = pm5.astype(jnp.int32) & 127
    j5f = j5.astype(jnp.float32)

    cand_d = [jnp.take_along_axis(sv, j5, axis=1, mode="promise_in_bounds")
              for sv in slices]
    lp0 = jnp.broadcast_to(lp_ref[0, :, :128], (tq, 128))
    lp1 = jnp.broadcast_to(lp_ref[0, :, 128:], (tq, 128))
    lab_a = jnp.take_along_axis(lp0, j5, axis=1, mode="promise_in_bounds")
    lab_b = jnp.take_along_axis(lp1, j5, axis=1, mode="promise_in_bounds")

    nc = _TOP_K * ns
    basef = (k * (4 * kb)).astype(jnp.float32)
    iota = jax.lax.broadcasted_iota(jnp.int32, (tq, nc), 1)
    sfl = (iota * 26215) >> 17
    wfl = iota - 5 * sfl
    tile_j = jnp.take_along_axis(j5f, wfl, axis=1,
                                 mode="promise_in_bounds")
    tile_la = jnp.take_along_axis(lab_a, wfl, axis=1,
                                  mode="promise_in_bounds")
    tile_lb = jnp.take_along_axis(lab_b, wfl, axis=1,
                                  mode="promise_in_bounds")
    sh_a = jnp.minimum(2 * sfl, 31)
    sh_b = jnp.maximum(2 * (sfl - 16), 0)
    lab_c = jnp.where(
        sfl < 16,
        jax.lax.shift_right_logical(tile_la, sh_a) & 3,
        jax.lax.shift_right_logical(tile_lb, sh_b) & 3).astype(jnp.float32)
    cand_p = (basef + (512.0 * sfl.astype(jnp.float32))
              + 4.0 * tile_j + lab_c)

    padd = jnp.full((tq, _NCAND - nc), jnp.inf, jnp.float32)
    padp = jnp.full((tq, _NCAND - nc), _BIGF, jnp.float32)
    d5_ref[...] = jnp.concatenate(cand_d + [padd], axis=1)[:, None, None, :]
    p5_ref[...] = jnp.concatenate([cand_p, padp], axis=1)[:, None, None, :]


def _phase2_body(d5_ref, p5_ref, out_ref, *, tq, nkb):
    nl = nkb * _NCAND
    dd = d5_ref[...].reshape(tq, nl)
    pp = p5_ref[...].reshape(tq, nl)

    ns2 = nl // 128
    items = [(dd[:, t * 128:(t + 1) * 128], pp[:, t * 128:(t + 1) * 128])
             for t in range(ns2)]
    d_slices = [it[0] for it in items]
    p_slices = [it[1] for it in items]
    while len(items) > 1:
        nxt = []
        for i in range(0, len(items) - 1, 2):
            (av, ap), (bv_, bp_) = items[i], items[i + 1]
            le = av <= bv_
            nxt.append((jnp.minimum(av, bv_), jnp.where(le, ap, bp_)))
        if len(items) % 2:
            nxt.append(items[-1])
        items = nxt
    rv, rp = items[0]

    lane = jax.lax.broadcasted_iota(jnp.int32, (tq, 128), 1).astype(
        jnp.float32)
    jcols = []
    for r in range(_TOP_K):
        mr = jnp.min(rv, axis=1, keepdims=True)
        pmr = jnp.min(jnp.where(rv == mr, rp, _BIGF), axis=1, keepdims=True)
        hit = rp == pmr
        jcols.append(jnp.min(jnp.where(hit, lane, 128.0), axis=1,
                             keepdims=True))
        if r < _TOP_K - 1:
            rv = jnp.where(hit, jnp.inf, rv)
    j5 = jnp.concatenate(jcols, axis=1).astype(jnp.int32)

    allv = jnp.concatenate(
        [jnp.take_along_axis(sv, j5, axis=1, mode="promise_in_bounds")
         for sv in d_slices], axis=1)
    allp = jnp.concatenate(
        [jnp.take_along_axis(sp_, j5, axis=1, mode="promise_in_bounds")
         for sp_ in p_slices], axis=1)
    np_ = []
    for r in range(_TOP_K):
        mr = jnp.min(allv, axis=1, keepdims=True)
        pmr = jnp.min(jnp.where(allv == mr, allp, _BIGF), axis=1,
                      keepdims=True)
        np_.append(pmr)
        if r < _TOP_K - 1:
            allv = jnp.where(allp == pmr, jnp.inf, allv)

    lab = jnp.concatenate(np_, axis=1).astype(jnp.int32) & 3
    cols = [jnp.sum((lab == c).astype(jnp.float32), axis=1, keepdims=True)
            for c in range(_NUM_CLASSES)]
    out_ref[...] = jnp.concatenate(cols, axis=1)


def kernel(x4, keys, labels):
    b, dmodel = x4.shape
    kn = keys.shape[0]
    tq = _TQ if b % _TQ == 0 else b
    kb = _KB
    kp = ((kn + kb - 1) // kb) * kb
    nkb = kp // kb
    nqt = b // tq
    ns = kb // 128

    q_sq = jnp.sum(x4 * x4, axis=1, keepdims=True)
    k_sq = jnp.sum(keys * keys, axis=1)
    pad = kp - kn
    k_sq_p = jnp.concatenate(
        [k_sq, jnp.full((pad,), jnp.inf, jnp.float32)]).reshape(nkb, 1, kb)
    keys_p = jnp.concatenate(
        [keys, jnp.zeros((pad, dmodel), jnp.float32)], axis=0)
    labels_p = jnp.concatenate(
        [labels, jnp.zeros((pad,), jnp.int32)]).reshape(nkb, ns, 128)
    shifts = (2 * (jnp.arange(16, dtype=jnp.int32)))[None, :, None]
    lp0 = jnp.sum(labels_p[:, :16] << shifts, axis=1, dtype=jnp.int32)
    lp1 = jnp.sum(labels_p[:, 16:] << shifts, axis=1, dtype=jnp.int32)
    lp = jnp.concatenate([lp0, lp1], axis=1).reshape(nkb, 1, 256)

    p1 = functools.partial(_phase1_body, tq=tq, kb=kb)
    d5, p5 = pl.pallas_call(
        p1,
        grid=(nkb, nqt),
        in_specs=[
            pl.BlockSpec((tq, dmodel), lambda k, q: (q, 0)),
            pl.BlockSpec((kb, dmodel), lambda k, q: (k, 0)),
            pl.BlockSpec((tq, 1), lambda k, q: (q, 0)),
            pl.BlockSpec((1, 1, kb), lambda k, q: (k, 0, 0)),
            pl.BlockSpec((1, 1, 256), lambda k, q: (k, 0, 0)),
        ],
        out_specs=[
            pl.BlockSpec((tq, 1, 1, _NCAND), lambda k, q: (q, k, 0, 0)),
            pl.BlockSpec((tq, 1, 1, _NCAND), lambda k, q: (q, k, 0, 0)),
        ],
        out_shape=[
            jax.ShapeDtypeStruct((b, nkb, 1, _NCAND), jnp.float32),
            jax.ShapeDtypeStruct((b, nkb, 1, _NCAND), jnp.float32),
        ],
        compiler_params=pltpu.CompilerParams(
            dimension_semantics=("parallel", "parallel")),
    )(x4, keys_p, q_sq, k_sq_p, lp)

    tq2 = 128 if b % 128 == 0 else b
    p2 = functools.partial(_phase2_body, tq=tq2, nkb=nkb)
    out = pl.pallas_call(
        p2,
        grid=(b // tq2,),
        in_specs=[
            pl.BlockSpec((tq2, nkb, 1, _NCAND), lambda q: (q, 0, 0, 0)),
            pl.BlockSpec((tq2, nkb, 1, _NCAND), lambda q: (q, 0, 0, 0)),
        ],
        out_specs=pl.BlockSpec((tq2, _NUM_CLASSES), lambda q: (q, 0)),
        out_shape=jax.ShapeDtypeStruct((b, _NUM_CLASSES), jnp.float32),
        compiler_params=pltpu.CompilerParams(
            dimension_semantics=("arbitrary",)),
    )(d5, p5)
    return out
